# baseline (speedup 1.0000x reference)
import functools
import math

import jax
import jax.numpy as jnp
from jax import lax
from jax.experimental import pallas as pl
from jax.experimental.pallas import tpu as pltpu
from jax.experimental.pallas import tpu_sc as plsc

WINDOW = 4
TOPK = 8
NEG_INF = float("-inf")


def _qk_body(x_ref, w_ref, b_ref, q_ref, k_ref):
    y = jnp.dot(x_ref[...], w_ref[...], preferred_element_type=jnp.float32)
    y = y + b_ref[...]
    d = q_ref.shape[1]
    q_ref[...] = y[:, :d].astype(jnp.bfloat16)
    k_ref[...] = y[:, d:].astype(jnp.bfloat16)


def _qk_proj(x, wqk, bqk, rb):
    n, d = x.shape
    return pl.pallas_call(
        _qk_body,
        grid=(n // rb,),
        in_specs=[
            pl.BlockSpec((rb, d), lambda i: (i, 0)),
            pl.BlockSpec((d, 2 * d), lambda i: (0, 0)),
            pl.BlockSpec((1, 2 * d), lambda i: (0, 0)),
        ],
        out_specs=[
            pl.BlockSpec((rb, d), lambda i: (i, 0)),
            pl.BlockSpec((rb, d), lambda i: (i, 0)),
        ],
        out_shape=[
            jax.ShapeDtypeStruct((n, d), jnp.bfloat16),
            jax.ShapeDtypeStruct((n, d), jnp.bfloat16),
        ],
    )(x, wqk, bqk)


def _vab_body(x_ref, w_ref, b_ref, v_ref, a_ref, bm_ref):
    y = jnp.dot(x_ref[...], w_ref[...], preferred_element_type=jnp.float32)
    y = y + b_ref[...]
    d = a_ref.shape[1]
    half = d // 2
    u = lax.bitcast_convert_type(y[:, :d], jnp.uint32)
    rnd = jnp.uint32(0x8000)
    ulo = (u[:, :half] + rnd) >> 16
    uhi = (u[:, half:d] + rnd) & jnp.uint32(0xFFFF0000)
    v_ref[...] = lax.bitcast_convert_type(ulo | uhi, jnp.int32)
    a_ref[...] = y[:, d:2 * d]
    bm_ref[...] = y[:, 2 * d:]


def _vab_proj(xb, wvab, bvab, rb):
    n, d = xb.shape
    return pl.pallas_call(
        _vab_body,
        grid=(n // rb,),
        in_specs=[
            pl.BlockSpec((rb, d), lambda i: (i, 0)),
            pl.BlockSpec((d, 3 * d), lambda i: (0, 0)),
            pl.BlockSpec((1, 3 * d), lambda i: (0, 0)),
        ],
        out_specs=[
            pl.BlockSpec((rb, d // 2), lambda i: (i, 0)),
            pl.BlockSpec((rb, d), lambda i: (i, 0)),
            pl.BlockSpec((rb, d), lambda i: (i, 0)),
        ],
        out_shape=[
            jax.ShapeDtypeStruct((n, d // 2), jnp.int32),
            jax.ShapeDtypeStruct((n, d), jnp.float32),
            jax.ShapeDtypeStruct((n, d), jnp.float32),
        ],
    )(xb, wvab, bvab)


def _silu_sum_body(a_ref, bc_ref, bp_ref, s_ref):
    i = pl.program_id(0)
    rb = a_ref.shape[0]
    a = a_ref[...]
    bc = bc_ref[...]
    bp = bp_ref[...]
    rows = lax.broadcasted_iota(jnp.int32, (rb, 1), 0)
    acc = jnp.zeros_like(a)
    for w in range(1, WINDOW + 1):
        sh = jnp.concatenate([bp[rb - w:], bc[:rb - w]], axis=0)
        sh = jnp.where((i == 0) & (rows < w), 0.0, sh)
        h = a + sh
        acc = acc + h * jax.nn.sigmoid(h)
    s_ref[...] = (acc * 0.25).astype(jnp.bfloat16)


def _silu_sum(a, bm, rb):
    n, d = a.shape
    return pl.pallas_call(
        _silu_sum_body,
        grid=(n // rb,),
        in_specs=[
            pl.BlockSpec((rb, d), lambda i: (i, 0)),
            pl.BlockSpec((rb, d), lambda i: (i, 0)),
            pl.BlockSpec((rb, d), lambda i: (jnp.maximum(i - 1, 0), 0)),
        ],
        out_specs=pl.BlockSpec((rb, d), lambda i: (i, 0)),
        out_shape=jax.ShapeDtypeStruct((n, d), jnp.bfloat16),
    )(a, bm, bm)


def _attn_body(q_ref, k_ref, pv_ref, pi_ref):
    i = pl.program_id(0)
    rb, d = q_ref.shape
    n = k_ref.shape[0]
    s = lax.dot_general(q_ref[...], k_ref[...],
                        (((1,), (1,)), ((), ())),
                        preferred_element_type=jnp.float32)
    s = s * (1.0 / math.sqrt(d))
    rows = i * rb + lax.broadcasted_iota(jnp.int32, (rb, n), 0)
    cols = lax.broadcasted_iota(jnp.int32, (rb, n), 1)
    s = jnp.where(cols > rows, NEG_INF, s)
    vals, idxs = [], []
    for _ in range(TOPK):
        m = jnp.max(s, axis=1, keepdims=True)
        c = jnp.min(jnp.where(s == m, cols, n), axis=1, keepdims=True)
        vals.append(m)
        idxs.append(c)
        s = jnp.where(cols == c, NEG_INF, s)
    v8 = jnp.concatenate(vals, axis=1)
    i8 = jnp.concatenate(idxs, axis=1)
    e = jnp.exp(v8 - v8[:, 0:1])
    pv_ref[...] = e / jnp.sum(e, axis=1, keepdims=True)
    pi_ref[...] = i8


def _attn_topk(q, k, rb):
    n, d = q.shape
    return pl.pallas_call(
        _attn_body,
        grid=(n // rb,),
        in_specs=[
            pl.BlockSpec((rb, d), lambda i: (i, 0)),
            pl.BlockSpec((n, d), lambda i: (0, 0)),
        ],
        out_specs=[
            pl.BlockSpec((rb, TOPK), lambda i: (i, 0)),
            pl.BlockSpec((rb, TOPK), lambda i: (i, 0)),
        ],
        out_shape=[
            jax.ShapeDtypeStruct((n, TOPK), jnp.float32),
            jax.ShapeDtypeStruct((n, TOPK), jnp.int32),
        ],
    )(q, k)


def _make_sc_ctx(n, d):
    info = plsc.get_sparse_core_info()
    nw = info.num_cores * info.num_subcores
    nc = info.num_cores
    rows_per_w = n // nw
    group = 2
    ngroups = rows_per_w // group
    gt = group * TOPK
    half = d // 2
    mesh = plsc.VectorSubcoreMesh(core_axis_name="c", subcore_axis_name="s")

    @functools.partial(
        pl.kernel,
        mesh=mesh,
        out_type=jax.ShapeDtypeStruct((n, half), jnp.int32),
        scratch_types=[
            pltpu.VMEM((rows_per_w * TOPK,), jnp.int32),
            pltpu.VMEM((rows_per_w * TOPK, 16), jnp.float32),
            pltpu.VMEM((gt, half), jnp.int32),
            pltpu.VMEM((gt, half), jnp.int32),
            pltpu.VMEM((group, half), jnp.int32),
            pltpu.VMEM((group, half), jnp.int32),
            pltpu.SemaphoreType.DMA,
            pltpu.SemaphoreType.DMA,
        ],
    )
    def sc_ctx(v_hbm, idx_hbm, p_hbm, out_hbm,
               idx_all, p_all, rows0, rows1, acc0, acc1, sem0, sem1):
        wid = lax.axis_index("s") * nc + lax.axis_index("c")
        wbase = wid * rows_per_w
        pltpu.sync_copy(idx_hbm.at[pl.ds(wbase * TOPK, rows_per_w * TOPK)],
                        idx_all)
        pltpu.sync_copy(p_hbm.at[pl.ds(wbase * TOPK, rows_per_w * TOPK)],
                        p_all)
        pltpu.async_copy(v_hbm.at[idx_all.at[pl.ds(0, gt)]], rows0, sem0)

        hi = jnp.int32(-65536)
        rnd = jnp.uint32(0x8000)

        def up(u):
            e = lax.bitcast_convert_type(u << 16, jnp.float32)
            o = lax.bitcast_convert_type(u & hi, jnp.float32)
            return e, o

        def down(acc_e, acc_o):
            ue = lax.bitcast_convert_type(acc_e, jnp.uint32)
            uo = lax.bitcast_convert_type(acc_o, jnp.uint32)
            ue = (ue + rnd) >> 16
            uo = (uo + rnd) & jnp.uint32(0xFFFF0000)
            return lax.bitcast_convert_type(ue | uo, jnp.int32)

        def compute(g, rows_v, acc_v):
            pbase = g * gt
            for r in range(group):
                pv = [p_all[pbase + r * TOPK + t, :] for t in range(TOPK)]

                def cbody(j, carry2, r=r, pv=pv, rows_v=rows_v, acc_v=acc_v):
                    for k2 in range(8):
                        cc = j * 128 + k2 * 16
                        e, o = up(rows_v[r * TOPK, pl.ds(cc, 16)])
                        acc_e = pv[0] * e
                        acc_o = pv[0] * o
                        for t in range(1, TOPK):
                            e, o = up(rows_v[r * TOPK + t, pl.ds(cc, 16)])
                            acc_e = acc_e + pv[t] * e
                            acc_o = acc_o + pv[t] * o
                        acc_v[r, pl.ds(cc, 16)] = down(acc_e, acc_o)
                    return carry2

                lax.fori_loop(0, half // 128, cbody, 0)
            pltpu.sync_copy(acc_v, out_hbm.at[pl.ds(wbase + g * group, group)])

        def body2(i, carry):
            g0 = 2 * i
            g1 = g0 + 1
            pltpu.async_copy(v_hbm.at[idx_all.at[pl.ds(g1 * gt, gt)]],
                             rows1, sem1)
            pltpu.make_async_copy(v_hbm.at[pl.ds(0, gt)], rows0, sem0).wait()
            compute(g0, rows0, acc0)

            @pl.when(g1 + 1 < ngroups)
            def _():
                pltpu.async_copy(
                    v_hbm.at[idx_all.at[pl.ds((g1 + 1) * gt, gt)]],
                    rows0, sem0)

            pltpu.make_async_copy(v_hbm.at[pl.ds(0, gt)], rows1, sem1).wait()
            compute(g1, rows1, acc1)
            return carry

        lax.fori_loop(0, ngroups // 2, body2, 0)

    return sc_ctx


def _out_body(s_ref, ctx_ref, wm2_ref, bm2_ref, wo1_ref, wo2_ref, bo_ref,
              o_ref):
    m = jnp.dot(s_ref[...], wm2_ref[...], preferred_element_type=jnp.float32)
    m = (m + bm2_ref[...]).astype(jnp.bfloat16)
    u = lax.bitcast_convert_type(ctx_ref[...], jnp.uint32)
    lo = lax.bitcast_convert_type(u << 16, jnp.float32)
    hi = lax.bitcast_convert_type(u & jnp.uint32(0xFFFF0000), jnp.float32)
    ctxb = jnp.concatenate([lo, hi], axis=1).astype(jnp.bfloat16)
    o = jnp.dot(m, wo1_ref[...], preferred_element_type=jnp.float32)
    o = o + jnp.dot(ctxb, wo2_ref[...], preferred_element_type=jnp.float32)
    o_ref[...] = o + bo_ref[...]


def _out_proj(s, ctx, wm2, bm2, wo1, wo2, bo, rb):
    n, d = s.shape
    return pl.pallas_call(
        _out_body,
        grid=(n // rb,),
        in_specs=[
            pl.BlockSpec((rb, d), lambda i: (i, 0)),
            pl.BlockSpec((rb, d // 2), lambda i: (i, 0)),
            pl.BlockSpec((d, d), lambda i: (0, 0)),
            pl.BlockSpec((1, d), lambda i: (0, 0)),
            pl.BlockSpec((d, d), lambda i: (0, 0)),
            pl.BlockSpec((d, d), lambda i: (0, 0)),
            pl.BlockSpec((1, d), lambda i: (0, 0)),
        ],
        out_specs=pl.BlockSpec((rb, d), lambda i: (i, 0)),
        out_shape=jax.ShapeDtypeStruct((n, d), jnp.float32),
    )(s, ctx, wm2, bm2, wo1, wo2, bo)


def kernel(mu, Wq, bq, Wk, bk, Wv, bv, Wm1, bm1, Wm2, bm2, Wo, bo):
    b, n, d = mu.shape
    x = mu[0]

    wqk = jnp.concatenate([Wq, Wk], axis=1).astype(jnp.bfloat16)
    bqk = jnp.concatenate([bq, bk])[None, :]
    wvab = jnp.concatenate([Wv, Wm1[:d], Wm1[d:]], axis=1).astype(jnp.bfloat16)
    bvab = jnp.concatenate([bv, bm1, jnp.zeros_like(bm1)])[None, :]
    xb = x.astype(jnp.bfloat16)

    q, k = _qk_proj(xb, wqk, bqk, rb=256)
    v, a, bmat = _vab_proj(xb, wvab, bvab, rb=128)
    s = _silu_sum(a, bmat, rb=256)
    probs, idx = _attn_topk(q, k, rb=256)

    sc_ctx = _make_sc_ctx(n // 2, d)
    pb = jnp.broadcast_to(probs.reshape(-1, 1), (n * TOPK, 16))
    idxf = idx.reshape(-1)
    h = n // 2 * TOPK
    ctx0 = sc_ctx(v, idxf[:h], pb[:h])
    ctx1 = sc_ctx(v, idxf[h:], pb[h:])

    wm2b = Wm2.astype(jnp.bfloat16)
    wo1b = Wo[:d].astype(jnp.bfloat16)
    wo2b = Wo[d:].astype(jnp.bfloat16)
    out0 = _out_proj(s[:n // 2], ctx0, wm2b, bm2[None, :],
                     wo1b, wo2b, bo[None, :], rb=256)
    out1 = _out_proj(s[n // 2:], ctx1, wm2b, bm2[None, :],
                     wo1b, wo2b, bo[None, :], rb=256)
    return jnp.concatenate([out0, out1], axis=0)[None]


# speedup vs baseline: 1.0616x; 1.0616x over previous
import functools
import math

import jax
import jax.numpy as jnp
from jax import lax
from jax.experimental import pallas as pl
from jax.experimental.pallas import tpu as pltpu
from jax.experimental.pallas import tpu_sc as plsc

WINDOW = 4
TOPK = 8
NEG_INF = float("-inf")


def _qk_body(x_ref, w_ref, b_ref, q_ref, k_ref):
    y = jnp.dot(x_ref[...], w_ref[...], preferred_element_type=jnp.float32)
    y = y + b_ref[...]
    d = q_ref.shape[1]
    q_ref[...] = y[:, :d].astype(jnp.bfloat16)
    k_ref[...] = y[:, d:].astype(jnp.bfloat16)


def _qk_proj(x, wqk, bqk, rb):
    n, d = x.shape
    return pl.pallas_call(
        _qk_body,
        grid=(n // rb,),
        in_specs=[
            pl.BlockSpec((rb, d), lambda i: (i, 0)),
            pl.BlockSpec((d, 2 * d), lambda i: (0, 0)),
            pl.BlockSpec((1, 2 * d), lambda i: (0, 0)),
        ],
        out_specs=[
            pl.BlockSpec((rb, d), lambda i: (i, 0)),
            pl.BlockSpec((rb, d), lambda i: (i, 0)),
        ],
        out_shape=[
            jax.ShapeDtypeStruct((n, d), jnp.bfloat16),
            jax.ShapeDtypeStruct((n, d), jnp.bfloat16),
        ],
    )(x, wqk, bqk)


def _vab_body(x_ref, w_ref, b_ref, v_ref, a_ref, bm_ref):
    y = jnp.dot(x_ref[...], w_ref[...], preferred_element_type=jnp.float32)
    y = y + b_ref[...]
    d = a_ref.shape[1]
    half = d // 2
    u = lax.bitcast_convert_type(y[:, :d], jnp.uint32)
    rnd = jnp.uint32(0x8000)
    ulo = (u[:, :half] + rnd) >> 16
    uhi = (u[:, half:d] + rnd) & jnp.uint32(0xFFFF0000)
    v_ref[...] = lax.bitcast_convert_type(ulo | uhi, jnp.int32)
    a_ref[...] = y[:, d:2 * d]
    bm_ref[...] = y[:, 2 * d:]


def _vab_proj(xb, wvab, bvab, rb):
    n, d = xb.shape
    return pl.pallas_call(
        _vab_body,
        grid=(n // rb,),
        in_specs=[
            pl.BlockSpec((rb, d), lambda i: (i, 0)),
            pl.BlockSpec((d, 3 * d), lambda i: (0, 0)),
            pl.BlockSpec((1, 3 * d), lambda i: (0, 0)),
        ],
        out_specs=[
            pl.BlockSpec((rb, d // 2), lambda i: (i, 0)),
            pl.BlockSpec((rb, d), lambda i: (i, 0)),
            pl.BlockSpec((rb, d), lambda i: (i, 0)),
        ],
        out_shape=[
            jax.ShapeDtypeStruct((n, d // 2), jnp.int32),
            jax.ShapeDtypeStruct((n, d), jnp.float32),
            jax.ShapeDtypeStruct((n, d), jnp.float32),
        ],
    )(xb, wvab, bvab)


def _silu_sum_body(a_ref, bc_ref, bp_ref, s_ref):
    i = pl.program_id(0)
    rb = a_ref.shape[0]
    a = a_ref[...]
    bc = bc_ref[...]
    bp = bp_ref[...]
    rows = lax.broadcasted_iota(jnp.int32, (rb, 1), 0)
    acc = jnp.zeros_like(a)
    for w in range(1, WINDOW + 1):
        sh = jnp.concatenate([bp[rb - w:], bc[:rb - w]], axis=0)
        sh = jnp.where((i == 0) & (rows < w), 0.0, sh)
        h = a + sh
        acc = acc + h * jax.nn.sigmoid(h)
    s_ref[...] = (acc * 0.25).astype(jnp.bfloat16)


def _silu_sum(a, bm, rb):
    n, d = a.shape
    return pl.pallas_call(
        _silu_sum_body,
        grid=(n // rb,),
        in_specs=[
            pl.BlockSpec((rb, d), lambda i: (i, 0)),
            pl.BlockSpec((rb, d), lambda i: (i, 0)),
            pl.BlockSpec((rb, d), lambda i: (jnp.maximum(i - 1, 0), 0)),
        ],
        out_specs=pl.BlockSpec((rb, d), lambda i: (i, 0)),
        out_shape=jax.ShapeDtypeStruct((n, d), jnp.bfloat16),
    )(a, bm, bm)


def _attn_body(q_ref, k_ref, pv_ref, pi_ref):
    i = pl.program_id(0)
    rb, d = q_ref.shape
    n = k_ref.shape[0]
    s = lax.dot_general(q_ref[...], k_ref[...],
                        (((1,), (1,)), ((), ())),
                        preferred_element_type=jnp.float32)
    s = s * (1.0 / math.sqrt(d))
    rows = i * rb + lax.broadcasted_iota(jnp.int32, (rb, n), 0)
    cols = lax.broadcasted_iota(jnp.int32, (rb, n), 1)
    s = jnp.where(cols > rows, NEG_INF, s)
    vals, idxs = [], []
    for _ in range(TOPK):
        m = jnp.max(s, axis=1, keepdims=True)
        c = jnp.min(jnp.where(s == m, cols, n), axis=1, keepdims=True)
        vals.append(m)
        idxs.append(c)
        s = jnp.where(cols == c, NEG_INF, s)
    v8 = jnp.concatenate(vals, axis=1)
    i8 = jnp.concatenate(idxs, axis=1)
    e = jnp.exp(v8 - v8[:, 0:1])
    pv_ref[...] = e / jnp.sum(e, axis=1, keepdims=True)
    pi_ref[...] = i8


def _attn_topk(q, k, rb):
    n, d = q.shape
    return pl.pallas_call(
        _attn_body,
        grid=(n // rb,),
        in_specs=[
            pl.BlockSpec((rb, d), lambda i: (i, 0)),
            pl.BlockSpec((n, d), lambda i: (0, 0)),
        ],
        out_specs=[
            pl.BlockSpec((rb, TOPK), lambda i: (i, 0)),
            pl.BlockSpec((rb, TOPK), lambda i: (i, 0)),
        ],
        out_shape=[
            jax.ShapeDtypeStruct((n, TOPK), jnp.float32),
            jax.ShapeDtypeStruct((n, TOPK), jnp.int32),
        ],
    )(q, k)


def _make_sc_ctx(n, d):
    info = plsc.get_sparse_core_info()
    nw = info.num_cores * info.num_subcores
    nc = info.num_cores
    rows_per_w = n // nw
    group = 2
    ngroups = rows_per_w // group
    gt = group * TOPK
    half = d // 2
    mesh = plsc.VectorSubcoreMesh(core_axis_name="c", subcore_axis_name="s")

    @functools.partial(
        pl.kernel,
        mesh=mesh,
        out_type=jax.ShapeDtypeStruct((n, half), jnp.int32),
        scratch_types=[
            pltpu.VMEM((rows_per_w * TOPK,), jnp.int32),
            pltpu.VMEM((rows_per_w * TOPK, 16), jnp.float32),
            pltpu.VMEM((gt, half), jnp.int32),
            pltpu.VMEM((gt, half), jnp.int32),
            pltpu.VMEM((group, half), jnp.int32),
            pltpu.VMEM((group, half), jnp.int32),
            pltpu.SemaphoreType.DMA,
            pltpu.SemaphoreType.DMA,
        ],
    )
    def sc_ctx(v_hbm, idx_hbm, p_hbm, out_hbm,
               idx_all, p_all, rows0, rows1, acc0, acc1, sem0, sem1):
        wid = lax.axis_index("s") * nc + lax.axis_index("c")
        wbase = wid * rows_per_w
        pltpu.sync_copy(idx_hbm.at[pl.ds(wbase * TOPK, rows_per_w * TOPK)],
                        idx_all)
        pltpu.sync_copy(p_hbm.at[pl.ds(wbase * TOPK, rows_per_w * TOPK)],
                        p_all)
        pltpu.async_copy(v_hbm.at[idx_all.at[pl.ds(0, gt)]], rows0, sem0)

        hi = jnp.int32(-65536)
        rnd = jnp.uint32(0x8000)

        def up(u):
            e = lax.bitcast_convert_type(u << 16, jnp.float32)
            o = lax.bitcast_convert_type(u & hi, jnp.float32)
            return e, o

        def down(acc_e, acc_o):
            ue = lax.bitcast_convert_type(acc_e, jnp.uint32)
            uo = lax.bitcast_convert_type(acc_o, jnp.uint32)
            ue = (ue + rnd) >> 16
            uo = (uo + rnd) & jnp.uint32(0xFFFF0000)
            return lax.bitcast_convert_type(ue | uo, jnp.int32)

        def compute(g, rows_v, acc_v):
            pbase = g * gt
            for r in range(group):
                pv = [p_all[pbase + r * TOPK + t, :] for t in range(TOPK)]

                def cbody(j, carry2, r=r, pv=pv, rows_v=rows_v, acc_v=acc_v):
                    for k2 in range(8):
                        cc = j * 128 + k2 * 16
                        e, o = up(rows_v[r * TOPK, pl.ds(cc, 16)])
                        acc_e = pv[0] * e
                        acc_o = pv[0] * o
                        for t in range(1, TOPK):
                            e, o = up(rows_v[r * TOPK + t, pl.ds(cc, 16)])
                            acc_e = acc_e + pv[t] * e
                            acc_o = acc_o + pv[t] * o
                        acc_v[r, pl.ds(cc, 16)] = down(acc_e, acc_o)
                    return carry2

                lax.fori_loop(0, half // 128, cbody, 0)
            pltpu.sync_copy(acc_v, out_hbm.at[pl.ds(wbase + g * group, group)])

        def body2(i, carry):
            g0 = 2 * i
            g1 = g0 + 1
            pltpu.async_copy(v_hbm.at[idx_all.at[pl.ds(g1 * gt, gt)]],
                             rows1, sem1)
            pltpu.make_async_copy(v_hbm.at[pl.ds(0, gt)], rows0, sem0).wait()
            compute(g0, rows0, acc0)

            @pl.when(g1 + 1 < ngroups)
            def _():
                pltpu.async_copy(
                    v_hbm.at[idx_all.at[pl.ds((g1 + 1) * gt, gt)]],
                    rows0, sem0)

            pltpu.make_async_copy(v_hbm.at[pl.ds(0, gt)], rows1, sem1).wait()
            compute(g1, rows1, acc1)
            return carry

        lax.fori_loop(0, ngroups // 2, body2, 0)

    return sc_ctx


def _out_body(s_ref, ctx_ref, wm2_ref, bm2_ref, wo1_ref, wo2_ref, bo_ref,
              o_ref):
    m = jnp.dot(s_ref[...], wm2_ref[...], preferred_element_type=jnp.float32)
    m = (m + bm2_ref[...]).astype(jnp.bfloat16)
    u = lax.bitcast_convert_type(ctx_ref[...], jnp.uint32)
    lo = lax.bitcast_convert_type(u << 16, jnp.float32)
    hi = lax.bitcast_convert_type(u & jnp.uint32(0xFFFF0000), jnp.float32)
    ctxb = jnp.concatenate([lo, hi], axis=1).astype(jnp.bfloat16)
    o = jnp.dot(m, wo1_ref[...], preferred_element_type=jnp.float32)
    o = o + jnp.dot(ctxb, wo2_ref[...], preferred_element_type=jnp.float32)
    o_ref[...] = o + bo_ref[...]


def _out_proj(s, ctx, wm2, bm2, wo1, wo2, bo, rb):
    n, d = s.shape
    return pl.pallas_call(
        _out_body,
        grid=(n // rb,),
        in_specs=[
            pl.BlockSpec((rb, d), lambda i: (i, 0)),
            pl.BlockSpec((rb, d // 2), lambda i: (i, 0)),
            pl.BlockSpec((d, d), lambda i: (0, 0)),
            pl.BlockSpec((1, d), lambda i: (0, 0)),
            pl.BlockSpec((d, d), lambda i: (0, 0)),
            pl.BlockSpec((d, d), lambda i: (0, 0)),
            pl.BlockSpec((1, d), lambda i: (0, 0)),
        ],
        out_specs=pl.BlockSpec((rb, d), lambda i: (i, 0)),
        out_shape=jax.ShapeDtypeStruct((n, d), jnp.float32),
    )(s, ctx, wm2, bm2, wo1, wo2, bo)


def kernel(mu, Wq, bq, Wk, bk, Wv, bv, Wm1, bm1, Wm2, bm2, Wo, bo):
    b, n, d = mu.shape
    x = mu[0]

    wqk = jnp.concatenate([Wq, Wk], axis=1).astype(jnp.bfloat16)
    bqk = jnp.concatenate([bq, bk])[None, :]
    wvab = jnp.concatenate([Wv, Wm1[:d], Wm1[d:]], axis=1).astype(jnp.bfloat16)
    bvab = jnp.concatenate([bv, bm1, jnp.zeros_like(bm1)])[None, :]
    xb = x.astype(jnp.bfloat16)

    q, k = _qk_proj(xb, wqk, bqk, rb=256)
    v, a, bmat = _vab_proj(xb, wvab, bvab, rb=128)
    s = _silu_sum(a, bmat, rb=256)
    probs, idx = _attn_topk(q, k, rb=256)

    sc_ctx = _make_sc_ctx(n, d)
    pb = jnp.broadcast_to(probs.reshape(-1, 1), (n * TOPK, 16))
    ctx = sc_ctx(v, idx.reshape(-1), pb)

    out = _out_proj(s, ctx,
                    Wm2.astype(jnp.bfloat16), bm2[None, :],
                    Wo[:d].astype(jnp.bfloat16), Wo[d:].astype(jnp.bfloat16),
                    bo[None, :], rb=256)
    return out[None]
